# baseline (speedup 1.0000x reference)
import jax
import jax.numpy as jnp
from jax.experimental import pallas as pl
from jax.experimental.pallas import tpu as pltpu

_ROWS_PER_STEP = 8


def _body(x_ref, f_ref, y_ref):
    xv = x_ref[...]
    fv = f_ref[...]
    masked = jnp.where(xv != 0, fv, 0.0)
    s = jnp.sum(masked, axis=1, keepdims=True)
    inv = jnp.where(s == 0.0, 1.0, 1.0 / s)
    y_ref[...] = masked * inv


def kernel(x, f):
    B, N = x.shape
    R = _ROWS_PER_STEP
    f2 = f.reshape(1, N)
    return pl.pallas_call(
        _body,
        grid=(B // R,),
        in_specs=[
            pl.BlockSpec((R, N), lambda i: (i, 0)),
            pl.BlockSpec((1, N), lambda i: (0, 0)),
        ],
        out_specs=pl.BlockSpec((R, N), lambda i: (i, 0)),
        out_shape=jax.ShapeDtypeStruct((B, N), x.dtype),
        compiler_params=pltpu.CompilerParams(
            dimension_semantics=("arbitrary",),
        ),
    )(x, f2)


# speedup vs baseline: 1.0252x; 1.0252x over previous
import jax
import jax.numpy as jnp
from jax.experimental import pallas as pl
from jax.experimental.pallas import tpu as pltpu

_R = 16
_CR = 2
_CH = _R // _CR


def _chunk_copy(hbm_ref, buf_ref, sems, step, slot, c, to_vmem):
    base = step * _R + c * _CR
    hbm_slice = hbm_ref.at[pl.ds(base, _CR), :]
    buf_slice = buf_ref.at[slot, pl.ds(c * _CR, _CR), :]
    if to_vmem:
        return pltpu.make_async_copy(hbm_slice, buf_slice, sems.at[slot])
    return pltpu.make_async_copy(buf_slice, hbm_slice, sems.at[slot])


def _body(x_hbm, f_ref, y_hbm, x_buf, y_buf, in_sems, out_sems):
    i = pl.program_id(0)
    n = pl.num_programs(0)
    s = jax.lax.rem(i, 2)
    s1 = jax.lax.rem(i + 1, 2)

    def start_in(step, slot):
        for c in range(_CH):
            _chunk_copy(x_hbm, x_buf, in_sems, step, slot, c, True).start()

    def wait_in(step, slot):
        for c in range(_CH):
            _chunk_copy(x_hbm, x_buf, in_sems, step, slot, c, True).wait()

    def start_out(step, slot):
        for c in range(_CH):
            _chunk_copy(y_hbm, y_buf, out_sems, step, slot, c, False).start()

    def wait_out(step, slot):
        for c in range(_CH):
            _chunk_copy(y_hbm, y_buf, out_sems, step, slot, c, False).wait()

    @pl.when(i == 0)
    def _():
        start_in(0, 0)

    @pl.when(i + 1 < n)
    def _():
        start_in(i + 1, s1)

    @pl.when(i >= 2)
    def _():
        wait_out(i - 2, s)

    wait_in(i, s)

    xv = x_buf[s]
    fv = f_ref[...]
    masked = jnp.where(xv != 0.0, fv, 0.0)
    ssum = jnp.sum(masked, axis=1, keepdims=True)
    inv = jnp.where(ssum == 0.0, 1.0, 1.0 / ssum)
    y_buf[s] = masked * inv

    start_out(i, s)

    @pl.when(i == n - 1)
    def _():
        wait_out(i - 1, s1)
        wait_out(i, s)


def kernel(x, f):
    B, N = x.shape
    f2 = f.reshape(1, N)
    return pl.pallas_call(
        _body,
        grid=(B // _R,),
        in_specs=[
            pl.BlockSpec(memory_space=pl.ANY),
            pl.BlockSpec((1, N), lambda i: (0, 0)),
        ],
        out_specs=pl.BlockSpec(memory_space=pl.ANY),
        out_shape=jax.ShapeDtypeStruct((B, N), x.dtype),
        scratch_shapes=[
            pltpu.VMEM((2, _R, N), jnp.float32),
            pltpu.VMEM((2, _R, N), jnp.float32),
            pltpu.SemaphoreType.DMA((2,)),
            pltpu.SemaphoreType.DMA((2,)),
        ],
        compiler_params=pltpu.CompilerParams(
            dimension_semantics=("arbitrary",),
        ),
    )(x, f2)


# speedup vs baseline: 2.2356x; 2.1806x over previous
import jax
import jax.numpy as jnp
from jax.experimental import pallas as pl
from jax.experimental.pallas import tpu as pltpu


def _p1(x_ref, f_ref, m_ref, s_ref):
    i = pl.program_id(0)
    xv = x_ref[...]
    fv = f_ref[...]
    nz = xv != 0.0
    m_ref[...] = nz.astype(jnp.int8)
    masked = jnp.where(nz, fv, 0.0)
    fb, b = masked.shape
    part = jnp.sum(masked.reshape(fb // 8, 8, b), axis=0)

    @pl.when(i == 0)
    def _():
        s_ref[...] = part

    @pl.when(i > 0)
    def _():
        s_ref[...] += part


def _p2(m_ref, f_ref, s_ref, y_ref):
    sv = jnp.sum(s_ref[...], axis=0, keepdims=True)
    inv = jnp.where(sv == 0.0, 1.0, 1.0 / sv)
    mv = m_ref[...] != 0
    fv = f_ref[...]
    y_ref[...] = jnp.where(mv, fv * inv, 0.0)


def _pick_fb(n):
    for fb in (800, 200, 40, 8):
        if n % fb == 0:
            return fb
    return n


def kernel(x, f):
    B, N = x.shape
    xt = x.T
    f2 = f.reshape(N, 1)
    FB = _pick_fb(N)
    grid = (N // FB,)
    cp = pltpu.CompilerParams(dimension_semantics=("arbitrary",))

    mask, s = pl.pallas_call(
        _p1,
        grid=grid,
        in_specs=[
            pl.BlockSpec((FB, B), lambda i: (i, 0)),
            pl.BlockSpec((FB, 1), lambda i: (i, 0)),
        ],
        out_specs=[
            pl.BlockSpec((FB, B), lambda i: (i, 0)),
            pl.BlockSpec((8, B), lambda i: (0, 0)),
        ],
        out_shape=[
            jax.ShapeDtypeStruct((N, B), jnp.int8),
            jax.ShapeDtypeStruct((8, B), jnp.float32),
        ],
        compiler_params=cp,
    )(xt, f2)

    yt = pl.pallas_call(
        _p2,
        grid=grid,
        in_specs=[
            pl.BlockSpec((FB, B), lambda i: (i, 0)),
            pl.BlockSpec((FB, 1), lambda i: (i, 0)),
            pl.BlockSpec((8, B), lambda i: (0, 0)),
        ],
        out_specs=pl.BlockSpec((FB, B), lambda i: (i, 0)),
        out_shape=jax.ShapeDtypeStruct((N, B), jnp.float32),
        compiler_params=cp,
    )(mask, f2, s)

    return yt.T


# speedup vs baseline: 2.7910x; 1.2484x over previous
import jax
import jax.numpy as jnp
from jax.experimental import pallas as pl
from jax.experimental.pallas import tpu as pltpu

_FB = 800
_W = _FB // 32
_WROWS = 32
_NBUF = 4


def _p1(x_hbm, f_ref, bm_hbm, s_ref, x_buf, m_buf, in_sems, out_sems):
    i = pl.program_id(0)
    n = pl.num_programs(0)

    def in_copy(step, slot):
        return pltpu.make_async_copy(
            x_hbm.at[pl.ds(step * _FB, _FB), :], x_buf.at[slot],
            in_sems.at[slot])

    def out_copy(step, slot):
        return pltpu.make_async_copy(
            m_buf.at[slot], bm_hbm.at[pl.ds(step * _WROWS, _WROWS), :],
            out_sems.at[slot])

    @pl.when(i == 0)
    def _():
        in_copy(0, 0).start()
        in_copy(1, 1).start()
        in_copy(2, 2).start()

    @pl.when(i + 3 < n)
    def _():
        in_copy(i + 3, jax.lax.rem(i + 3, _NBUF)).start()

    oslot = jax.lax.rem(i, 2)

    @pl.when(i >= 2)
    def _():
        out_copy(i - 2, oslot).wait()

    slot = jax.lax.rem(i, _NBUF)
    in_copy(i, slot).wait()

    xv = x_buf[slot]
    nz = xv != 0.0
    fv = f_ref[...]
    b = xv.shape[1]
    masked = jnp.where(nz, fv, 0.0)
    part = jnp.sum(masked.reshape(_FB // 8, 8, b), axis=0)

    @pl.when(i == 0)
    def _():
        s_ref[...] = part

    @pl.when(i > 0)
    def _():
        s_ref[...] += part

    kvec = jax.lax.broadcasted_iota(jnp.int32, (_W, 32, b), 1)
    bits = nz.astype(jnp.int32).reshape(_W, 32, b)
    m_buf[oslot, 0:_W, :] = jnp.sum(bits << kvec, axis=1)

    out_copy(i, oslot).start()

    @pl.when(i == n - 1)
    def _():
        out_copy(i - 1, jax.lax.rem(i - 1, 2)).wait()
        out_copy(i, oslot).wait()


def _p2(bm_hbm, f_ref, s_ref, y_hbm, m_buf, y_buf, in_sems, out_sems):
    i = pl.program_id(0)
    n = pl.num_programs(0)

    def in_copy(step, slot):
        return pltpu.make_async_copy(
            bm_hbm.at[pl.ds(step * _WROWS, _WROWS), :], m_buf.at[slot],
            in_sems.at[slot])

    def out_copy(step, slot):
        return pltpu.make_async_copy(
            y_buf.at[slot], y_hbm.at[pl.ds(step * _FB, _FB), :],
            out_sems.at[slot])

    @pl.when(i == 0)
    def _():
        in_copy(0, 0).start()
        in_copy(1, 1).start()
        in_copy(2, 2).start()

    @pl.when(i + 3 < n)
    def _():
        in_copy(i + 3, jax.lax.rem(i + 3, _NBUF)).start()

    oslot = jax.lax.rem(i, _NBUF)

    @pl.when(i >= _NBUF)
    def _():
        out_copy(i - _NBUF, oslot).wait()

    slot = jax.lax.rem(i, _NBUF)
    in_copy(i, slot).wait()

    sv = jnp.sum(s_ref[...], axis=0, keepdims=True)
    inv = jnp.where(sv == 0.0, 1.0, 1.0 / sv)
    fv = f_ref[...]
    scale = fv * inv
    wv = m_buf[slot][0:_W, :]
    b = wv.shape[1]
    kvec = jax.lax.broadcasted_iota(jnp.int32, (_W, 32, b), 1)
    expand = jnp.broadcast_to(wv.reshape(_W, 1, b), (_W, 32, b))
    mv = ((expand >> kvec) & 1).reshape(_FB, b) != 0
    y_buf[oslot] = jnp.where(mv, scale, 0.0)

    out_copy(i, oslot).start()

    @pl.when(i == n - 1)
    def _():
        @pl.when(n >= 4)
        def _():
            out_copy(i - 3, jax.lax.rem(i - 3, _NBUF)).wait()

        @pl.when(n >= 3)
        def _():
            out_copy(i - 2, jax.lax.rem(i - 2, _NBUF)).wait()

        @pl.when(n >= 2)
        def _():
            out_copy(i - 1, jax.lax.rem(i - 1, _NBUF)).wait()

        out_copy(i, oslot).wait()


def kernel(x, f):
    B, N = x.shape
    xt = x.T
    nsteps = N // _FB
    f2 = f.reshape(N, 1)
    cp = pltpu.CompilerParams(dimension_semantics=("arbitrary",))

    bitmap, s = pl.pallas_call(
        _p1,
        grid=(nsteps,),
        in_specs=[
            pl.BlockSpec(memory_space=pl.ANY),
            pl.BlockSpec((_FB, 1), lambda i: (i, 0)),
        ],
        out_specs=[
            pl.BlockSpec(memory_space=pl.ANY),
            pl.BlockSpec((8, B), lambda i: (0, 0)),
        ],
        out_shape=[
            jax.ShapeDtypeStruct((nsteps * _WROWS, B), jnp.int32),
            jax.ShapeDtypeStruct((8, B), jnp.float32),
        ],
        scratch_shapes=[
            pltpu.VMEM((_NBUF, _FB, B), jnp.float32),
            pltpu.VMEM((2, _WROWS, B), jnp.int32),
            pltpu.SemaphoreType.DMA((_NBUF,)),
            pltpu.SemaphoreType.DMA((2,)),
        ],
        compiler_params=cp,
    )(xt, f2)

    yt = pl.pallas_call(
        _p2,
        grid=(nsteps,),
        in_specs=[
            pl.BlockSpec(memory_space=pl.ANY),
            pl.BlockSpec((_FB, 1), lambda i: (i, 0)),
            pl.BlockSpec((8, B), lambda i: (0, 0)),
        ],
        out_specs=pl.BlockSpec(memory_space=pl.ANY),
        out_shape=jax.ShapeDtypeStruct((N, B), jnp.float32),
        scratch_shapes=[
            pltpu.VMEM((_NBUF, _WROWS, B), jnp.int32),
            pltpu.VMEM((_NBUF, _FB, B), jnp.float32),
            pltpu.SemaphoreType.DMA((_NBUF,)),
            pltpu.SemaphoreType.DMA((_NBUF,)),
        ],
        compiler_params=cp,
    )(bitmap, f2, s)

    return yt.T
